# baseline (speedup 1.0000x reference)
import functools

import jax
import jax.numpy as jnp
from jax import lax
from jax.experimental import pallas as pl
from jax.experimental.pallas import tpu as pltpu
from jax.experimental.pallas import tpu_sc as plsc

EPS = 1e-6
NC = 2
NS = 16
NW = NC * NS

CHUNK = 256
DMA_ROWS = 128
LN_ROWS = 1024


def _make_gather(n_rows, padded):
    rows_per_w = n_rows // NW
    assert rows_per_w * NW == n_rows
    n_pairs = rows_per_w // (2 * CHUNK)
    assert n_pairs * 2 * CHUNK == rows_per_w

    mesh = plsc.VectorSubcoreMesh(core_axis_name="c", subcore_axis_name="s")

    @functools.partial(
        pl.kernel,
        out_type=jax.ShapeDtypeStruct((n_rows, padded), jnp.float32),
        mesh=mesh,
        scratch_types=[
            pltpu.VMEM((2 * CHUNK,), jnp.int32),
            pltpu.VMEM((CHUNK, padded), jnp.float32),
            pltpu.VMEM((CHUNK, padded), jnp.float32),
            pltpu.SemaphoreType.DMA,
            pltpu.SemaphoreType.DMA,
        ],
    )
    def kern(x_ref, tab_ref, out_ref, idx_v, rows_a, rows_b, semg, semo):
        wid = lax.axis_index("s") * NC + lax.axis_index("c")

        def pair_body(pi, carry):
            base = wid * rows_per_w + pi * (2 * CHUNK)
            pltpu.sync_copy(x_ref.at[pl.ds(base, 2 * CHUNK)], idx_v)
            gathers = []
            for half, buf in ((0, rows_a), (1, rows_b)):
                hs = []
                for j in range(CHUNK // DMA_ROWS):
                    off = half * CHUNK + j * DMA_ROWS
                    hs.append(pltpu.async_copy(
                        tab_ref.at[idx_v.at[pl.ds(off, DMA_ROWS)]],
                        buf.at[pl.ds(j * DMA_ROWS, DMA_ROWS)],
                        semg,
                    ))
                gathers.append(hs)
            outs = []
            for half, buf in ((0, rows_a), (1, rows_b)):
                for h in gathers[half]:
                    h.wait()
                outs.append(pltpu.async_copy(
                    buf, out_ref.at[pl.ds(base + half * CHUNK, CHUNK)], semo))
            for o in outs:
                o.wait()
            return carry

        lax.fori_loop(0, n_pairs, pair_body, 0)

    return kern


def _ln_block(raw_ref, al_ref, be_ref, out_ref, *, hidden):
    y = raw_ref[:, :hidden]
    mean = jnp.sum(y, axis=1, keepdims=True) * (1.0 / hidden)
    d = y - mean
    var = jnp.sum(d * d, axis=1, keepdims=True) * (1.0 / (hidden - 1))
    sigma = jnp.sqrt(var) + EPS
    out_ref[...] = al_ref[0] * (d / sigma + be_ref[0])


def kernel(x, table, alpha, beta):
    b, l = x.shape
    vocab, hidden = table.shape
    n_rows = b * l
    padded = 2 * hidden
    x_flat = x.reshape(-1).astype(jnp.int32)
    table_p = jnp.pad(table, ((0, 0), (0, padded - hidden)))
    raw = _make_gather(n_rows, padded)(x_flat, table_p)

    ln = pl.pallas_call(
        functools.partial(_ln_block, hidden=hidden),
        grid=(n_rows // LN_ROWS,),
        in_specs=[
            pl.BlockSpec((LN_ROWS, padded), lambda i: (i, 0)),
            pl.BlockSpec((1, hidden), lambda i: (0, 0)),
            pl.BlockSpec((1, hidden), lambda i: (0, 0)),
        ],
        out_specs=pl.BlockSpec((LN_ROWS, hidden), lambda i: (i, 0)),
        out_shape=jax.ShapeDtypeStruct((n_rows, hidden), jnp.float32),
    )
    out = ln(raw, alpha.reshape(1, hidden), beta.reshape(1, hidden))
    return out.reshape(b, l, hidden)
